# Initial kernel scaffold; baseline (speedup 1.0000x reference)
import jax
import jax.numpy as jnp
from jax.experimental import pallas as pl


def kernel(seq1, seq2, diff, W2, b2, a2, sparse):
    raise NotImplementedError("write your pallas kernel here")


# speedup vs baseline: 1.9864x; 1.9864x over previous
import jax
import jax.numpy as jnp
from jax.experimental import pallas as pl
from jax.experimental.pallas import tpu as pltpu

N = 10000
D = 128
H = 128
BM = 400


def _gcn_kernel(diff_ref, seq1_ref, seq2_ref, w_ref, b_ref, a_ref,
                hmask_ref, h2_ref, f_scratch):
    i = pl.program_id(0)

    @pl.when(i == 0)
    def _compute_features():
        w_t = w_ref[...].T
        f2 = jnp.dot(seq2_ref[...], w_t, preferred_element_type=jnp.float32)
        f1 = jnp.dot(seq1_ref[...], w_t, preferred_element_type=jnp.float32)
        f_scratch[:, :H] = f2.astype(jnp.bfloat16)
        f_scratch[:, H:] = f1.astype(jnp.bfloat16)

    acc = jnp.dot(diff_ref[...].astype(jnp.bfloat16), f_scratch[...],
                  preferred_element_type=jnp.float32)
    out = acc + b_ref[0:1, :]
    a = a_ref[0, 0]
    out = jnp.where(out >= 0.0, out, a * out)
    out = jnp.maximum(out, 0.0)
    hmask_ref[...] = out[:, :H]
    h2_ref[...] = out[:, H:]


def kernel(seq1, seq2, diff, W2, b2, a2, sparse):
    del sparse
    s1 = seq1.reshape(N, D)
    s2 = seq2.reshape(N, D)
    dmat = diff.reshape(N, N)
    b = jnp.concatenate([b2, b2]).reshape(1, 2 * H)
    a = a2.reshape(1, 1)

    grid = (N // BM,)
    h_mask, h_2 = pl.pallas_call(
        _gcn_kernel,
        grid=grid,
        in_specs=[
            pl.BlockSpec((BM, N), lambda i: (i, 0)),
            pl.BlockSpec((N, D), lambda i: (0, 0)),
            pl.BlockSpec((N, D), lambda i: (0, 0)),
            pl.BlockSpec((H, D), lambda i: (0, 0)),
            pl.BlockSpec((1, 2 * H), lambda i: (0, 0)),
            pl.BlockSpec((1, 1), lambda i: (0, 0)),
        ],
        out_specs=[
            pl.BlockSpec((BM, H), lambda i: (i, 0)),
            pl.BlockSpec((BM, H), lambda i: (i, 0)),
        ],
        out_shape=[
            jax.ShapeDtypeStruct((N, H), jnp.float32),
            jax.ShapeDtypeStruct((N, H), jnp.float32),
        ],
        scratch_shapes=[pltpu.VMEM((N, 2 * H), jnp.bfloat16)],
    )(dmat, s1, s2, W2, b, a)

    return (h_mask[None, ...], h_2[None, ...])
